# baseline (speedup 1.0000x reference)
import jax
import jax.numpy as jnp
from jax.experimental import pallas as pl
from jax.experimental.pallas import tpu as pltpu

NODE_DIM = 1024
NUM_TYPES = 16
BATCH = 256
NCHUNK = 2


def _mixer_body(at_ref, z_ref, g_ref, bt_ref, w1_ref, b1_ref, w2_ref, b2_ref,
                o_ref, azn_ref, acc_ref):
    g = pl.program_id(0)
    c = pl.program_id(1)
    m = jnp.minimum(g, NUM_TYPES - 1)
    k = jnp.maximum(g - 1, 0)
    half = c * (NUM_TYPES // 2)

    part = at_ref[m, half] * z_ref[:, half, :]
    for j in range(1, NUM_TYPES // 2):
        part = part + at_ref[m, half + j] * z_ref[:, half + j, :]
    acc = part + jnp.where(c == 0, jnp.zeros_like(part), acc_ref[...])
    acc_ref[...] = acc
    mu = jnp.mean(acc, axis=1, keepdims=True)
    m2 = jnp.mean(acc * acc, axis=1, keepdims=True)
    azn_ref[g % 2] = (acc - mu) * jax.lax.rsqrt(m2 - mu * mu + 1e-5) \
        * g_ref[0, :] + bt_ref[0, :]

    azn = azn_ref[k % 2]
    h = jnp.dot(azn, w1_ref[0], preferred_element_type=jnp.float32) \
        + b1_ref[0, 0, :]
    h = jnp.where(h > 0, h, jnp.exp(h) - 1.0)
    mlp = jnp.dot(h, w2_ref[0], preferred_element_type=jnp.float32)
    base = jnp.where(c == 0, azn + b2_ref[0, 0, :], o_ref[:, k, :])
    o_ref[:, k, :] = base + mlp


def kernel(z, A, gamma, beta, W1, b1, W2, b2):
    K = NUM_TYPES
    d = NODE_DIM
    B = z.shape[0] // K
    hc = 2 * d // NCHUNK
    zb = z.reshape(B, K, d)
    at = A.T
    g2 = gamma.reshape(1, d)
    bt2 = beta.reshape(1, d)
    b1r = b1.reshape(K, 1, 2 * d)
    b2r = b2.reshape(K, 1, d)

    def wk(g, c):
        return jnp.maximum(g - 1, 0)

    out = pl.pallas_call(
        _mixer_body,
        grid=(K + 1, NCHUNK),
        in_specs=[
            pl.BlockSpec(memory_space=pltpu.SMEM),
            pl.BlockSpec((B, K, d), lambda g, c: (0, 0, 0)),
            pl.BlockSpec((1, d), lambda g, c: (0, 0)),
            pl.BlockSpec((1, d), lambda g, c: (0, 0)),
            pl.BlockSpec((1, d, hc), lambda g, c: (wk(g, c), 0, c)),
            pl.BlockSpec((1, 1, hc), lambda g, c: (wk(g, c), 0, c)),
            pl.BlockSpec((1, hc, d), lambda g, c: (wk(g, c), c, 0)),
            pl.BlockSpec((1, 1, d), lambda g, c: (wk(g, c), 0, 0)),
        ],
        out_specs=pl.BlockSpec((B, K, d), lambda g, c: (0, 0, 0)),
        out_shape=jax.ShapeDtypeStruct((B, K, d), jnp.float32),
        scratch_shapes=[
            pltpu.VMEM((2, B, d), jnp.float32),
            pltpu.VMEM((B, d), jnp.float32),
        ],
        compiler_params=pltpu.CompilerParams(
            dimension_semantics=("arbitrary", "arbitrary"),
        ),
    )(at, zb, g2, bt2, W1, b1r, W2, b2r)
    return out.reshape(B * K, d)
